# baseline (speedup 1.0000x reference)
import functools

import jax
import jax.numpy as jnp
from jax import lax
from jax.experimental import pallas as pl
from jax.experimental.pallas import tpu as pltpu
from jax.experimental.pallas import tpu_sc as plsc

_B = 64
_N = 49152
_NC = 2
_NS = 16
_NW = _NC * _NS
_CHUNK = _N // _NW
_GROUPS = _CHUNK // 16


def _sc_mode_body(x_hbm, out_hbm, x_v, mode_v, hist):
    wid = lax.axis_index("s") * _NC + lax.axis_index("c")
    base = wid * _CHUNK
    pltpu.sync_copy(x_hbm.at[:, pl.ds(base, _CHUNK)], x_v)
    lane = lax.iota(jnp.int32, 16)
    zeros16 = jnp.zeros((16,), jnp.int32)

    def zero_body(j, c):
        hist[pl.ds(j * 16, 16)] = zeros16
        return c

    lax.fori_loop(0, 256, zero_body, 0)

    def group_body(g, c):
        off = g * 16
        best = zeros16
        for b in range(_B):
            vi = x_v[b, pl.ds(off, 16)].astype(jnp.int32)
            addr = vi * 16 + lane
            h1 = plsc.load_gather(hist, [addr]) + 1
            plsc.store_scatter(hist, [addr], h1)
            best = jnp.maximum(best, h1 * 256 + (255 - vi))
        for b in range(_B):
            vi = x_v[b, pl.ds(off, 16)].astype(jnp.int32)
            plsc.store_scatter(hist, [vi * 16 + lane], zeros16)
        mode_v[pl.ds(off, 16)] = (255 - (best & 255)).astype(jnp.float32)
        return c

    lax.fori_loop(0, _GROUPS, group_body, 0)
    pltpu.sync_copy(mode_v, out_hbm.at[pl.ds(base, _CHUNK)])


_sc_mode = functools.partial(
    pl.kernel,
    out_type=jax.ShapeDtypeStruct((_N,), jnp.float32),
    mesh=plsc.VectorSubcoreMesh(core_axis_name="c", subcore_axis_name="s"),
    scratch_types=[
        pltpu.VMEM((_B, _CHUNK), jnp.float32),
        pltpu.VMEM((_CHUNK,), jnp.float32),
        pltpu.VMEM((4096,), jnp.int32),
    ],
    compiler_params=pltpu.CompilerParams(needs_layout_passes=False),
)(_sc_mode_body)


def _expand_body(x_ref, m_ref, bg_ref, fg_ref):
    x = x_ref[...]
    bg = jnp.broadcast_to(m_ref[...], x.shape)
    bg_ref[...] = bg
    fg_ref[...] = jnp.abs(x - bg)


def kernel(input):
    B, C, H, W = input.shape
    N = C * H * W
    x2 = input.reshape(B, N)
    mode = _sc_mode(x2)
    m2 = mode.reshape(1, N)
    BLK = 2048
    bg, fg = pl.pallas_call(
        _expand_body,
        grid=(N // BLK,),
        in_specs=[pl.BlockSpec((B, BLK), lambda i: (0, i)),
                  pl.BlockSpec((1, BLK), lambda i: (0, i))],
        out_specs=[pl.BlockSpec((B, BLK), lambda i: (0, i)),
                   pl.BlockSpec((B, BLK), lambda i: (0, i))],
        out_shape=[jax.ShapeDtypeStruct((B, N), jnp.float32),
                   jax.ShapeDtypeStruct((B, N), jnp.float32)],
    )(x2, m2)
    return bg.reshape(input.shape), fg.reshape(input.shape)


# speedup vs baseline: 1.2801x; 1.2801x over previous
import functools

import jax
import jax.numpy as jnp
from jax import lax
from jax.experimental import pallas as pl
from jax.experimental.pallas import tpu as pltpu
from jax.experimental.pallas import tpu_sc as plsc

_B = 64
_N = 49152
_NC = 2
_NS = 16
_NW = _NC * _NS
_CHUNK = _N // _NW
_GROUPS = _CHUNK // 16


_ILV = 4


def _sc_mode_body(x_hbm, out_hbm, x_v, mode_v, addr_v, h0, h1, h2, h3):
    wid = lax.axis_index("s") * _NC + lax.axis_index("c")
    base = wid * _CHUNK
    pltpu.sync_copy(x_hbm.at[:, pl.ds(base, _CHUNK)], x_v)
    lane = lax.iota(jnp.int32, 16)
    zeros16 = jnp.zeros((16,), jnp.int32)
    hists = (h0, h1, h2, h3)

    def zero_body(j, c):
        for h in hists:
            h[pl.ds(j * 16, 16)] = zeros16
        return c

    lax.fori_loop(0, 256, zero_body, 0)

    def group_body(g, c):
        off0 = g * (16 * _ILV)
        for b in range(_B):
            vis = [x_v[b, pl.ds(off0 + 16 * k, 16)].astype(jnp.int32)
                   for k in range(_ILV)]
            for k in range(_ILV):
                addr_v[pl.ds((b * _ILV + k) * 16, 16)] = vis[k] * 16 + lane
        bests = [zeros16] * _ILV
        for b in range(_B):
            addrs = [addr_v[pl.ds((b * _ILV + k) * 16, 16)]
                     for k in range(_ILV)]
            cnts = [plsc.load_gather(hists[k], [addrs[k]]) + 1
                    for k in range(_ILV)]
            for k in range(_ILV):
                plsc.store_scatter(hists[k], [addrs[k]], cnts[k])
            for k in range(_ILV):
                key = jnp.left_shift(cnts[k], 8) + (
                    255 - jnp.right_shift(addrs[k], 4))
                bests[k] = jnp.maximum(bests[k], key)
        for b in range(_B):
            addrs = [addr_v[pl.ds((b * _ILV + k) * 16, 16)]
                     for k in range(_ILV)]
            for k in range(_ILV):
                plsc.store_scatter(hists[k], [addrs[k]], zeros16)
        for k in range(_ILV):
            mode_v[pl.ds(off0 + 16 * k, 16)] = (
                255 - (bests[k] & 255)).astype(jnp.float32)
        return c

    lax.fori_loop(0, _GROUPS // _ILV, group_body, 0)
    pltpu.sync_copy(mode_v, out_hbm.at[pl.ds(base, _CHUNK)])


_sc_mode = functools.partial(
    pl.kernel,
    out_type=jax.ShapeDtypeStruct((_N,), jnp.float32),
    mesh=plsc.VectorSubcoreMesh(core_axis_name="c", subcore_axis_name="s"),
    scratch_types=[
        pltpu.VMEM((_B, _CHUNK), jnp.float32),
        pltpu.VMEM((_CHUNK,), jnp.float32),
        pltpu.VMEM((_B * _ILV * 16,), jnp.int32),
        pltpu.VMEM((4096,), jnp.int32),
        pltpu.VMEM((4096,), jnp.int32),
        pltpu.VMEM((4096,), jnp.int32),
        pltpu.VMEM((4096,), jnp.int32),
    ],
    compiler_params=pltpu.CompilerParams(needs_layout_passes=False),
)(_sc_mode_body)


def _expand_body(x_ref, m_ref, bg_ref, fg_ref):
    x = x_ref[...]
    bg = jnp.broadcast_to(m_ref[...], x.shape)
    bg_ref[...] = bg
    fg_ref[...] = jnp.abs(x - bg)


def kernel(input):
    B, C, H, W = input.shape
    N = C * H * W
    x2 = input.reshape(B, N)
    mode = _sc_mode(x2)
    m2 = mode.reshape(1, N)
    BLK = 2048
    bg, fg = pl.pallas_call(
        _expand_body,
        grid=(N // BLK,),
        in_specs=[pl.BlockSpec((B, BLK), lambda i: (0, i)),
                  pl.BlockSpec((1, BLK), lambda i: (0, i))],
        out_specs=[pl.BlockSpec((B, BLK), lambda i: (0, i)),
                   pl.BlockSpec((B, BLK), lambda i: (0, i))],
        out_shape=[jax.ShapeDtypeStruct((B, N), jnp.float32),
                   jax.ShapeDtypeStruct((B, N), jnp.float32)],
    )(x2, m2)
    return bg.reshape(input.shape), fg.reshape(input.shape)


# speedup vs baseline: 1.4288x; 1.1162x over previous
import functools

import jax
import jax.numpy as jnp
from jax import lax
from jax.experimental import pallas as pl
from jax.experimental.pallas import tpu as pltpu
from jax.experimental.pallas import tpu_sc as plsc

_B = 64
_N = 49152
_NC = 2
_NS = 16
_NW = _NC * _NS
_CHUNK = _N // _NW
_GROUPS = _CHUNK // 16


_ILV = 4


def _sc_mode_body(x_hbm, out_hbm, x_v, mode_v, addr_v, h0, h1, h2, h3):
    wid = lax.axis_index("s") * _NC + lax.axis_index("c")
    base = wid * _CHUNK
    pltpu.sync_copy(x_hbm.at[:, pl.ds(base, _CHUNK)], x_v)
    lane = lax.iota(jnp.int32, 16)
    zeros16 = jnp.zeros((16,), jnp.int32)
    hists = (h0, h1, h2, h3)

    def zero_body(j, c):
        for h in hists:
            h[pl.ds(j * 16, 16)] = zeros16
        return c

    lax.fori_loop(0, 256, zero_body, 0)

    def group_body(g, c):
        off0 = g * (16 * _ILV)
        gbase = g * 256
        for b in range(_B):
            vis = [x_v[b, pl.ds(off0 + 16 * k, 16)].astype(jnp.int32)
                   for k in range(_ILV)]
            for k in range(_ILV):
                addr_v[pl.ds((b * _ILV + k) * 16, 16)] = vis[k] * 16 + lane
        bests = [zeros16] * _ILV
        for b in range(_B):
            addrs = [addr_v[pl.ds((b * _ILV + k) * 16, 16)]
                     for k in range(_ILV)]
            ents = [jnp.maximum(plsc.load_gather(hists[k], [addrs[k]]),
                                gbase) + 1
                    for k in range(_ILV)]
            for k in range(_ILV):
                plsc.store_scatter(hists[k], [addrs[k]], ents[k])
            for k in range(_ILV):
                key = jnp.left_shift(ents[k] - gbase, 12) + (4095 - addrs[k])
                bests[k] = jnp.maximum(bests[k], key)
        for k in range(_ILV):
            mode_v[pl.ds(off0 + 16 * k, 16)] = jnp.right_shift(
                4095 - (bests[k] & 4095), 4).astype(jnp.float32)
        return c

    lax.fori_loop(0, _GROUPS // _ILV, group_body, 0)
    pltpu.sync_copy(mode_v, out_hbm.at[pl.ds(base, _CHUNK)])


_sc_mode = functools.partial(
    pl.kernel,
    out_type=jax.ShapeDtypeStruct((_N,), jnp.float32),
    mesh=plsc.VectorSubcoreMesh(core_axis_name="c", subcore_axis_name="s"),
    scratch_types=[
        pltpu.VMEM((_B, _CHUNK), jnp.float32),
        pltpu.VMEM((_CHUNK,), jnp.float32),
        pltpu.VMEM((_B * _ILV * 16,), jnp.int32),
        pltpu.VMEM((4096,), jnp.int32),
        pltpu.VMEM((4096,), jnp.int32),
        pltpu.VMEM((4096,), jnp.int32),
        pltpu.VMEM((4096,), jnp.int32),
    ],
    compiler_params=pltpu.CompilerParams(needs_layout_passes=False),
)(_sc_mode_body)


def _expand_body(x_ref, m_ref, bg_ref, fg_ref):
    x = x_ref[...]
    bg = jnp.broadcast_to(m_ref[...], x.shape)
    bg_ref[...] = bg
    fg_ref[...] = jnp.abs(x - bg)


def kernel(input):
    B, C, H, W = input.shape
    N = C * H * W
    x2 = input.reshape(B, N)
    mode = _sc_mode(x2)
    m2 = mode.reshape(1, N)
    BLK = 2048
    bg, fg = pl.pallas_call(
        _expand_body,
        grid=(N // BLK,),
        in_specs=[pl.BlockSpec((B, BLK), lambda i: (0, i)),
                  pl.BlockSpec((1, BLK), lambda i: (0, i))],
        out_specs=[pl.BlockSpec((B, BLK), lambda i: (0, i)),
                   pl.BlockSpec((B, BLK), lambda i: (0, i))],
        out_shape=[jax.ShapeDtypeStruct((B, N), jnp.float32),
                   jax.ShapeDtypeStruct((B, N), jnp.float32)],
    )(x2, m2)
    return bg.reshape(input.shape), fg.reshape(input.shape)


# speedup vs baseline: 2.8177x; 1.9721x over previous
import functools

import jax
import jax.numpy as jnp
from jax import lax
from jax.experimental import pallas as pl
from jax.experimental.pallas import tpu as pltpu
from jax.experimental.pallas import tpu_sc as plsc

_B = 64
_R = 384
_W = 128
_N = _R * _W
_NC = 2
_NS = 16
_NW = _NC * _NS
_RPW = _R // _NW
_WIN = 16
_CHUNK = _RPW * _W
_GROUPS = _CHUNK // 16
_ILV = 4
_HB = _B // 2


def _sc_mode_body(x_hbm, out_hbm, x_v, mode_v, addr_v, best_v,
                  h0, h1, h2, h3, sem):
    wid = lax.axis_index("s") * _NC + lax.axis_index("c")
    r0 = wid * _RPW
    astart = pl.multiple_of((r0 // 8) * 8, 8)
    s0 = r0 - astart

    lane = lax.iota(jnp.int32, 16)
    zeros16 = jnp.zeros((16,), jnp.int32)
    hists = (h0, h1, h2, h3)

    def zero_body(j, c):
        for h in hists:
            h[pl.ds(j * 16, 16)] = zeros16
        return c

    lax.fori_loop(0, 256, zero_body, 0)

    for half in range(2):
        copies = [
            pltpu.make_async_copy(
                x_hbm.at[half * _HB + i, pl.ds(astart, _WIN), :],
                x_v.at[pl.ds(i * _WIN, _WIN), :], sem)
            for i in range(_HB)
        ]
        for c in copies:
            c.start()
        for c in copies:
            c.wait()

        def group_body(g, c, half=half):
            off0 = g * (16 * _ILV)
            gbase = g * 256
            row = s0 + jnp.right_shift(off0, 7)
            l0 = off0 & 127
            for i in range(_HB):
                vis = [x_v[i * _WIN + row, pl.ds(l0 + 16 * k, 16)
                           ].astype(jnp.int32)
                       for k in range(_ILV)]
                for k in range(_ILV):
                    addr_v[pl.ds((i * _ILV + k) * 16, 16)] = \
                        vis[k] * 16 + lane
            if half == 0:
                bests = [zeros16] * _ILV
            else:
                bests = [best_v[pl.ds((g * _ILV + k) * 16, 16)]
                         for k in range(_ILV)]
            for i in range(_HB):
                addrs = [addr_v[pl.ds((i * _ILV + k) * 16, 16)]
                         for k in range(_ILV)]
                hs = [plsc.load_gather(hists[k], [addrs[k]])
                      for k in range(_ILV)]
                if half == 0:
                    ents = [jnp.maximum(h, gbase) + 1 for h in hs]
                else:
                    ents = [jnp.where(jnp.right_shift(h, 8) == g, h, gbase)
                            + 1 for h in hs]
                for k in range(_ILV):
                    plsc.store_scatter(hists[k], [addrs[k]], ents[k])
                for k in range(_ILV):
                    key = jnp.left_shift(ents[k] - gbase, 12) + (
                        4095 - addrs[k])
                    bests[k] = jnp.maximum(bests[k], key)
            if half == 0:
                for k in range(_ILV):
                    best_v[pl.ds((g * _ILV + k) * 16, 16)] = bests[k]
            else:
                for k in range(_ILV):
                    mode_v[pl.ds(off0 + 16 * k, 16)] = \
                        jnp.right_shift(4095 - (bests[k] & 4095),
                                        4).astype(jnp.float32)
            return c

        lax.fori_loop(0, _GROUPS // _ILV, group_body, 0)

    pltpu.sync_copy(mode_v, out_hbm.at[pl.ds(wid * _CHUNK, _CHUNK)])


_sc_mode = functools.partial(
    pl.kernel,
    out_type=jax.ShapeDtypeStruct((_N,), jnp.float32),
    mesh=plsc.VectorSubcoreMesh(core_axis_name="c", subcore_axis_name="s"),
    scratch_types=[
        pltpu.VMEM((_HB * _WIN, _W), jnp.float32),
        pltpu.VMEM((_CHUNK,), jnp.float32),
        pltpu.VMEM((_HB * _ILV * 16,), jnp.int32),
        pltpu.VMEM((_GROUPS * 16,), jnp.int32),
        pltpu.VMEM((4096,), jnp.int32),
        pltpu.VMEM((4096,), jnp.int32),
        pltpu.VMEM((4096,), jnp.int32),
        pltpu.VMEM((4096,), jnp.int32),
        pltpu.SemaphoreType.DMA,
    ],
    compiler_params=pltpu.CompilerParams(needs_layout_passes=False),
)(_sc_mode_body)


def _expand_body(x_ref, m_ref, bg_ref, fg_ref):
    x = x_ref[...]
    bg = jnp.broadcast_to(m_ref[...][None], x.shape)
    bg_ref[...] = bg
    fg_ref[...] = jnp.abs(x - bg)


def kernel(input):
    B, C, H, W = input.shape
    x3 = input.reshape(B, C * H, W)
    mode = _sc_mode(x3).reshape(_R, _W)
    RB = 48
    bg, fg = pl.pallas_call(
        _expand_body,
        grid=(_R // RB,),
        in_specs=[pl.BlockSpec((B, RB, W), lambda i: (0, i, 0)),
                  pl.BlockSpec((RB, W), lambda i: (i, 0))],
        out_specs=[pl.BlockSpec((B, RB, W), lambda i: (0, i, 0)),
                   pl.BlockSpec((B, RB, W), lambda i: (0, i, 0))],
        out_shape=[jax.ShapeDtypeStruct((B, C * H, W), jnp.float32),
                   jax.ShapeDtypeStruct((B, C * H, W), jnp.float32)],
    )(x3, mode)
    return bg.reshape(input.shape), fg.reshape(input.shape)
